# Initial kernel scaffold; baseline (speedup 1.0000x reference)
import jax
import jax.numpy as jnp
from jax.experimental import pallas as pl


def kernel(inputs, Wr, W1, b1, W2, b2):
    raise NotImplementedError("write your pallas kernel here")


# speedup vs baseline: 4.5788x; 4.5788x over previous
import jax
import jax.numpy as jnp
from jax.experimental import pallas as pl
from jax.experimental.pallas import tpu as pltpu

B, S, D = 2, 2048, 768
E = 8
FF = 1024
N = B * S
TT = 256


def _moe_dense_body(x_ref, wr_ref, w1_ref, b1_ref, w2_ref, b2_ref, out_ref):
    x = x_ref[...]
    logits = jnp.dot(x, wr_ref[...], preferred_element_type=jnp.float32)
    probs = jax.nn.softmax(logits, axis=-1)
    i1 = jnp.argmax(probs, axis=-1)[:, None]
    p1 = jnp.max(probs, axis=-1, keepdims=True)
    cols = jax.lax.broadcasted_iota(jnp.int32, probs.shape, 1)
    masked = jnp.where(cols == i1, -jnp.inf, probs)
    i2 = jnp.argmax(masked, axis=-1)[:, None]
    p2 = jnp.max(masked, axis=-1, keepdims=True)
    e2 = jnp.exp(p2 - p1)
    g1 = 1.0 / (1.0 + e2)
    g2 = e2 / (1.0 + e2)
    acc = jnp.zeros_like(x)
    for e in range(E):
        ge = jnp.where(i1 == e, g1, 0.0) + jnp.where(i2 == e, g2, 0.0)
        h = jnp.dot(x, w1_ref[e], preferred_element_type=jnp.float32) + b1_ref[e]
        h = h * 0.5 * (1.0 + jax.lax.erf(h * 0.7071067811865476))
        y = jnp.dot(h, w2_ref[e], preferred_element_type=jnp.float32) + b2_ref[e]
        acc = acc + y * ge
    out_ref[...] = acc


def kernel(inputs, Wr, W1, b1, W2, b2):
    x = inputs.reshape(N, D)
    out = pl.pallas_call(
        _moe_dense_body,
        grid=(N // TT,),
        in_specs=[
            pl.BlockSpec((TT, D), lambda i: (i, 0)),
            pl.BlockSpec((D, E), lambda i: (0, 0)),
            pl.BlockSpec((E, D, FF), lambda i: (0, 0, 0)),
            pl.BlockSpec((E, FF), lambda i: (0, 0)),
            pl.BlockSpec((E, FF, D), lambda i: (0, 0, 0)),
            pl.BlockSpec((E, D), lambda i: (0, 0)),
        ],
        out_specs=pl.BlockSpec((TT, D), lambda i: (i, 0)),
        out_shape=jax.ShapeDtypeStruct((N, D), jnp.float32),
        compiler_params=pltpu.CompilerParams(
            dimension_semantics=("arbitrary",),
        ),
    )(x, Wr, W1, b1, W2, b2)
    return out.reshape(B, S, D)
